# baseline (speedup 1.0000x reference)
import functools

import jax
import jax.numpy as jnp
from jax import lax
from jax.experimental import pallas as pl
from jax.experimental.pallas import tpu as pltpu
from jax.experimental.pallas import tpu_sc as plsc

_R = 2048
_BINS = 1024
_DUMP = 1016
_TILES = 16
_L = 16


def _sc_body(y_hbm, winv_hbm, k_hbm, y_v, cnt_v, winv_v, big_v, kv_v,
             shared_all):
    cid = lax.axis_index("c")
    sid = lax.axis_index("s")
    n = y_hbm.shape[0]
    per = n // _TILES

    @pl.when(cid == 0)
    def _():
        base = sid * per
        pltpu.sync_copy(y_hbm.at[pl.ds(base, per)], y_v)

        def zb(b, _):
            cnt_v[pl.ds(b * _L, _L)] = jnp.zeros((_L,), jnp.float32)
            return _
        lax.fori_loop(0, _BINS // _L, zb, 0)

        ones16 = jnp.ones((_L,), jnp.float32)

        def hist(j, _):
            yv = y_v[pl.ds(j * _L, _L)]
            msk = yv >= 0
            safe = jnp.where(msk, yv, _DUMP)
            plsc.addupdate_scatter(cnt_v, [safe], ones16, mask=msk)
            return _
        lax.fori_loop(0, per // _L, hist, 0)
        pltpu.sync_copy(cnt_v, shared_all.at[sid])
        plsc.subcore_barrier()
        pltpu.sync_copy(shared_all, big_v)

        def red(b, _):
            acc = big_v[0, pl.ds(b * _L, _L)]
            for r in range(1, _TILES):
                acc = acc + big_v[r, pl.ds(b * _L, _L)]
            cnt_v[pl.ds(b * _L, _L)] = acc
            return _
        lax.fori_loop(0, _BINS // _L, red, 0)

        def wloop(j, _):
            yv = y_v[pl.ds(j * _L, _L)]
            msk = yv >= 0
            safe = jnp.where(msk, yv, 0)
            cnt = plsc.load_gather(cnt_v, [safe])
            wv = jnp.where(msk, 1.0 / cnt, 0.0)
            winv_v[pl.ds(j * _L, _L)] = wv
            return _
        lax.fori_loop(0, per // _L, wloop, 0)
        base = sid * per
        pltpu.sync_copy(winv_v, winv_hbm.at[pl.ds(base, per)])

        @pl.when(sid == 0)
        def _():
            def kloop(b, acc):
                cnt = cnt_v[pl.ds(b * _L, _L)]
                binid = lax.iota(jnp.int32, _L) + b * _L
                live = (cnt > 0.0) & (binid < 1000)
                return acc + jnp.where(live, 1.0, 0.0)
            kacc = lax.fori_loop(0, _BINS // _L, kloop, jnp.zeros((_L,), jnp.float32))
            ktot = jnp.sum(kacc)
            kv_v[...] = jnp.full((_L,), ktot, dtype=jnp.float32)
            pltpu.sync_copy(kv_v, k_hbm)


def _sc_weights(pseudo_label):
    n = pseudo_label.shape[0]
    mesh = plsc.VectorSubcoreMesh(core_axis_name="c", subcore_axis_name="s")
    fn = functools.partial(
        pl.kernel,
        mesh=mesh,
        compiler_params=pltpu.CompilerParams(needs_layout_passes=False),
        out_type=(
            jax.ShapeDtypeStruct((n,), jnp.float32),
            jax.ShapeDtypeStruct((_L,), jnp.float32),
        ),
        scratch_types=[
            pltpu.VMEM((n // _TILES,), jnp.int32),
            pltpu.VMEM((_BINS,), jnp.float32),
            pltpu.VMEM((n // _TILES,), jnp.float32),
            pltpu.VMEM((_TILES, _BINS), jnp.float32),
            pltpu.VMEM((_L,), jnp.float32),
            pltpu.VMEM_SHARED((_TILES, _BINS), jnp.float32),
        ],
    )(_sc_body)
    return fn(pseudo_label)


def _tc_body(c_ref, y_ref, w_ref, k_ref, out_ref):
    i = pl.program_id(0)
    x = c_ref[...].astype(jnp.float32)
    r, cnum = x.shape
    m = jnp.max(x, axis=1, keepdims=True)
    se = jnp.sum(jnp.exp(x - m), axis=1, keepdims=True)
    lse = jnp.log(se) + m
    y = y_ref[...]
    col = lax.broadcasted_iota(jnp.int32, (r, cnum), 1)
    oh = col == y
    picked = jnp.sum(jnp.where(oh, x, 0.0), axis=1, keepdims=True)
    nll = lse - picked
    part = jnp.sum(w_ref[...] * nll)

    @pl.when(i == 0)
    def _():
        out_ref[...] = jnp.zeros_like(out_ref)

    out_ref[...] += jnp.full((1, 1), part, dtype=jnp.float32)

    @pl.when(i == pl.num_programs(0) - 1)
    def _():
        k = k_ref[0, 0]
        tot = out_ref[0, 0]
        out_ref[...] = jnp.full(
            (1, 1), jnp.where(k > 0.0, tot / jnp.where(k > 0.0, k, 1.0), 0.0),
            dtype=jnp.float32)


def kernel(c, pseudo_label):
    n, cnum = c.shape
    cb = c.astype(jnp.bfloat16)
    y2d = pseudo_label.reshape(n, 1).astype(jnp.int32)
    winv, kvec = _sc_weights(pseudo_label.astype(jnp.int32))
    winv2d = winv.reshape(n, 1)
    kk = kvec[:1].reshape(1, 1)
    out = pl.pallas_call(
        _tc_body,
        grid=(n // _R,),
        in_specs=[
            pl.BlockSpec((_R, cnum), lambda i: (i, 0)),
            pl.BlockSpec((_R, 1), lambda i: (i, 0)),
            pl.BlockSpec((_R, 1), lambda i: (i, 0)),
            pl.BlockSpec((1, 1), lambda i: (0, 0)),
        ],
        out_specs=pl.BlockSpec((1, 1), lambda i: (0, 0)),
        out_shape=jax.ShapeDtypeStruct((1, 1), jnp.float32),
    )(cb, y2d, winv2d, kk)
    return out[0, 0]


# speedup vs baseline: 1.0009x; 1.0009x over previous
import functools

import jax
import jax.numpy as jnp
from jax import lax
from jax.experimental import pallas as pl
from jax.experimental.pallas import tpu as pltpu
from jax.experimental.pallas import tpu_sc as plsc

_R = 2048
_BINS = 1024
_NCLU = 1000
_DUMP = 1016
_TILES = 16
_L = 16
_SLC = 128
_MTILES = _BINS // _SLC


def _sc_body(y_hbm, winv_hbm, y_v, cnt_v, winv_v, red_v, slc_v,
             shared_all, shared_glob):
    cid = lax.axis_index("c")
    sid = lax.axis_index("s")
    n = y_hbm.shape[0]
    per = n // _TILES

    @pl.when(cid == 0)
    def _():
        base = sid * per
        pltpu.sync_copy(y_hbm.at[pl.ds(base, per)], y_v)

        def zb(b, _):
            cnt_v[pl.ds(b * _L, _L)] = jnp.zeros((_L,), jnp.float32)
            return _
        lax.fori_loop(0, _BINS // _L, zb, 0)

        ones16 = jnp.ones((_L,), jnp.float32)

        def hist(j, _):
            yv = y_v[pl.ds(j * _L, _L)]
            msk = yv >= 0
            safe = jnp.where(msk, yv, _DUMP)
            plsc.addupdate_scatter(cnt_v, [safe], ones16, mask=msk)
            return _
        lax.fori_loop(0, per // _L, hist, 0)
        pltpu.sync_copy(cnt_v, shared_all.at[sid])
        plsc.subcore_barrier()

        @pl.when(sid < _MTILES)
        def _():
            pltpu.sync_copy(shared_all.at[:, pl.ds(sid * _SLC, _SLC)], red_v)
            for b in range(_SLC // _L):
                acc = red_v[0, pl.ds(b * _L, _L)]
                for r in range(1, _TILES):
                    acc = acc + red_v[r, pl.ds(b * _L, _L)]
                slc_v[pl.ds(b * _L, _L)] = acc
            pltpu.sync_copy(slc_v, shared_glob.at[pl.ds(sid * _SLC, _SLC)])
        plsc.subcore_barrier()
        pltpu.sync_copy(shared_glob, cnt_v)

        def kloop(b, acc):
            cnt = cnt_v[pl.ds(b * _L, _L)]
            binid = lax.iota(jnp.int32, _L) + b * _L
            live = (cnt > 0.0) & (binid < _NCLU)
            return acc + jnp.where(live, 1.0, 0.0)
        kacc = lax.fori_loop(0, _BINS // _L, kloop,
                             jnp.zeros((_L,), jnp.float32))
        ktot = jnp.full((_L,), jnp.sum(kacc), dtype=jnp.float32)
        kinv = jnp.where(ktot > 0.0, 1.0 / ktot, 0.0)

        def wloop(j, _):
            yv = y_v[pl.ds(j * _L, _L)]
            msk = yv >= 0
            safe = jnp.where(msk, yv, 0)
            cnt = plsc.load_gather(cnt_v, [safe])
            wv = jnp.where(msk, kinv / cnt, 0.0)
            winv_v[pl.ds(j * _L, _L)] = wv
            return _
        lax.fori_loop(0, per // _L, wloop, 0)
        pltpu.sync_copy(winv_v, winv_hbm.at[pl.ds(base, per)])


def _sc_weights(pseudo_label):
    n = pseudo_label.shape[0]
    mesh = plsc.VectorSubcoreMesh(core_axis_name="c", subcore_axis_name="s")
    fn = functools.partial(
        pl.kernel,
        mesh=mesh,
        compiler_params=pltpu.CompilerParams(needs_layout_passes=False),
        out_type=jax.ShapeDtypeStruct((n,), jnp.float32),
        scratch_types=[
            pltpu.VMEM((n // _TILES,), jnp.int32),
            pltpu.VMEM((_BINS,), jnp.float32),
            pltpu.VMEM((n // _TILES,), jnp.float32),
            pltpu.VMEM((_TILES, _SLC), jnp.float32),
            pltpu.VMEM((_SLC,), jnp.float32),
            pltpu.VMEM_SHARED((_TILES, _BINS), jnp.float32),
            pltpu.VMEM_SHARED((_BINS,), jnp.float32),
        ],
    )(_sc_body)
    return fn(pseudo_label)


def _tc_body(c_ref, y_ref, w_ref, out_ref):
    i = pl.program_id(0)
    x = c_ref[...].astype(jnp.float32)
    r, cnum = x.shape
    m = jnp.max(x, axis=1, keepdims=True)
    se = jnp.sum(jnp.exp(x - m), axis=1, keepdims=True)
    lse = jnp.log(se) + m
    y = y_ref[...]
    col = lax.broadcasted_iota(jnp.int32, (r, cnum), 1)
    oh = col == y
    picked = jnp.sum(jnp.where(oh, x, 0.0), axis=1, keepdims=True)
    nll = lse - picked
    part = jnp.sum(w_ref[...] * nll)

    @pl.when(i == 0)
    def _():
        out_ref[...] = jnp.zeros_like(out_ref)

    out_ref[...] += jnp.full((1, 1), part, dtype=jnp.float32)


def kernel(c, pseudo_label):
    n, cnum = c.shape
    cb = c.astype(jnp.bfloat16)
    y2d = pseudo_label.reshape(n, 1).astype(jnp.int32)
    winv = _sc_weights(pseudo_label.astype(jnp.int32))
    winv2d = winv.reshape(n, 1)
    out = pl.pallas_call(
        _tc_body,
        grid=(n // _R,),
        in_specs=[
            pl.BlockSpec((_R, cnum), lambda i: (i, 0)),
            pl.BlockSpec((_R, 1), lambda i: (i, 0)),
            pl.BlockSpec((_R, 1), lambda i: (i, 0)),
        ],
        out_specs=pl.BlockSpec((1, 1), lambda i: (0, 0)),
        out_shape=jax.ShapeDtypeStruct((1, 1), jnp.float32),
    )(cb, y2d, winv2d)
    return out[0, 0]


# speedup vs baseline: 1.1145x; 1.1135x over previous
import functools

import jax
import jax.numpy as jnp
from jax import lax
from jax.experimental import pallas as pl
from jax.experimental.pallas import tpu as pltpu
from jax.experimental.pallas import tpu_sc as plsc

_R = 2048
_BINS = 1024
_NCLU = 1000
_DUMP = 1016
_TILES = 16
_L = 16
_SLC = 128
_MTILES = _BINS // _SLC


def _sc_body(y_hbm, winv_hbm, y_v, cnt_v, winv_v, red_v, slc_v,
             shared_all, shared_glob):
    cid = lax.axis_index("c")
    sid = lax.axis_index("s")
    n = y_hbm.shape[0]
    per = n // _TILES

    @pl.when(cid == 0)
    def _():
        base = sid * per
        pltpu.sync_copy(y_hbm.at[pl.ds(base, per)], y_v)

        def zb(b, _):
            cnt_v[pl.ds(b * _L, _L)] = jnp.zeros((_L,), jnp.float32)
            return _
        lax.fori_loop(0, _BINS // _L, zb, 0)

        ones16 = jnp.ones((_L,), jnp.float32)

        def hist(j, _):
            yv = y_v[pl.ds(j * _L, _L)]
            msk = yv >= 0
            safe = jnp.where(msk, yv, _DUMP)
            plsc.addupdate_scatter(cnt_v, [safe], ones16, mask=msk)
            return _
        lax.fori_loop(0, per // _L, hist, 0)
        pltpu.sync_copy(cnt_v, shared_all.at[sid])
        plsc.subcore_barrier()

        @pl.when(sid < _MTILES)
        def _():
            pltpu.sync_copy(shared_all.at[:, pl.ds(sid * _SLC, _SLC)], red_v)
            for b in range(_SLC // _L):
                acc = red_v[0, pl.ds(b * _L, _L)]
                for r in range(1, _TILES):
                    acc = acc + red_v[r, pl.ds(b * _L, _L)]
                slc_v[pl.ds(b * _L, _L)] = acc
            pltpu.sync_copy(slc_v, shared_glob.at[pl.ds(sid * _SLC, _SLC)])
        plsc.subcore_barrier()
        pltpu.sync_copy(shared_glob, cnt_v)

        def kloop(b, acc):
            cnt = cnt_v[pl.ds(b * _L, _L)]
            binid = lax.iota(jnp.int32, _L) + b * _L
            live = (cnt > 0.0) & (binid < _NCLU)
            return acc + jnp.where(live, 1.0, 0.0)
        kacc = lax.fori_loop(0, _BINS // _L, kloop,
                             jnp.zeros((_L,), jnp.float32))
        ktot = jnp.full((_L,), jnp.sum(kacc), dtype=jnp.float32)
        kinv = jnp.where(ktot > 0.0, 1.0 / ktot, 0.0)

        def wloop(j, _):
            yv = y_v[pl.ds(j * _L, _L)]
            msk = yv >= 0
            safe = jnp.where(msk, yv, 0)
            cnt = plsc.load_gather(cnt_v, [safe])
            wv = jnp.where(msk, kinv / cnt, 0.0)
            winv_v[pl.ds(j * _L, _L)] = wv
            return _
        lax.fori_loop(0, per // _L, wloop, 0)
        pltpu.sync_copy(winv_v, winv_hbm.at[pl.ds(base, per)])


def _sc_weights(pseudo_label):
    n = pseudo_label.shape[0]
    mesh = plsc.VectorSubcoreMesh(core_axis_name="c", subcore_axis_name="s")
    fn = functools.partial(
        pl.kernel,
        mesh=mesh,
        compiler_params=pltpu.CompilerParams(needs_layout_passes=False),
        out_type=jax.ShapeDtypeStruct((n,), jnp.float32),
        scratch_types=[
            pltpu.VMEM((n // _TILES,), jnp.int32),
            pltpu.VMEM((_BINS,), jnp.float32),
            pltpu.VMEM((n // _TILES,), jnp.float32),
            pltpu.VMEM((_TILES, _SLC), jnp.float32),
            pltpu.VMEM((_SLC,), jnp.float32),
            pltpu.VMEM_SHARED((_TILES, _BINS), jnp.float32),
            pltpu.VMEM_SHARED((_BINS,), jnp.float32),
        ],
    )(_sc_body)
    return fn(pseudo_label)


def _tc_body(c_ref, y_ref, w_ref, out_ref):
    i = pl.program_id(0)
    x = c_ref[...]
    r, cnum = x.shape
    m = jnp.max(x, axis=1, keepdims=True)
    se = jnp.sum(jnp.exp(x - m), axis=1, keepdims=True)
    lse = jnp.log(se) + m
    y = y_ref[...]
    col = lax.broadcasted_iota(jnp.int32, (r, cnum), 1)
    oh = col == y
    picked = jnp.sum(jnp.where(oh, x, 0.0), axis=1, keepdims=True)
    nll = lse - picked
    part = jnp.sum(w_ref[...] * nll)

    @pl.when(i == 0)
    def _():
        out_ref[...] = jnp.zeros_like(out_ref)

    out_ref[...] += jnp.full((1, 1), part, dtype=jnp.float32)


def kernel(c, pseudo_label):
    n, cnum = c.shape
    y2d = pseudo_label.reshape(n, 1).astype(jnp.int32)
    winv = _sc_weights(pseudo_label.astype(jnp.int32))
    winv2d = winv.reshape(n, 1)
    out = pl.pallas_call(
        _tc_body,
        grid=(n // _R,),
        in_specs=[
            pl.BlockSpec((_R, cnum), lambda i: (i, 0)),
            pl.BlockSpec((_R, 1), lambda i: (i, 0)),
            pl.BlockSpec((_R, 1), lambda i: (i, 0)),
        ],
        out_specs=pl.BlockSpec((1, 1), lambda i: (0, 0)),
        out_shape=jax.ShapeDtypeStruct((1, 1), jnp.float32),
    )(c, y2d, winv2d)
    return out[0, 0]


# speedup vs baseline: 1.1150x; 1.0005x over previous
import functools

import jax
import jax.numpy as jnp
from jax import lax
from jax.experimental import pallas as pl
from jax.experimental.pallas import tpu as pltpu
from jax.experimental.pallas import tpu_sc as plsc

_R = 2048
_BINS = 1024
_NCLU = 1000
_DUMP = 1016
_TILES = 16
_L = 16
_SLC = 128
_MTILES = _BINS // _SLC


def _sc_body(y_hbm, winv_hbm, y_v, cnt_v, winv_v, red_v, slc_v,
             shared_all, shared_glob):
    cid = lax.axis_index("c")
    sid = lax.axis_index("s")
    n = y_hbm.shape[0]
    per = n // _TILES

    @pl.when(cid == 0)
    def _():
        base = sid * per
        pltpu.sync_copy(y_hbm.at[pl.ds(base, per)], y_v)

        def zb(b, _):
            cnt_v[pl.ds(b * _L, _L)] = jnp.zeros((_L,), jnp.float32)
            return _
        lax.fori_loop(0, _BINS // _L, zb, 0)

        ones16 = jnp.ones((_L,), jnp.float32)

        def hist(j, _):
            yv = y_v[pl.ds(j * _L, _L)]
            msk = yv >= 0
            safe = jnp.where(msk, yv, _DUMP)
            plsc.addupdate_scatter(cnt_v, [safe], ones16, mask=msk)
            return _
        lax.fori_loop(0, per // _L, hist, 0)
        pltpu.sync_copy(cnt_v, shared_all.at[sid])
        plsc.subcore_barrier()

        @pl.when(sid < _MTILES)
        def _():
            pltpu.sync_copy(shared_all.at[:, pl.ds(sid * _SLC, _SLC)], red_v)
            for b in range(_SLC // _L):
                acc = red_v[0, pl.ds(b * _L, _L)]
                for r in range(1, _TILES):
                    acc = acc + red_v[r, pl.ds(b * _L, _L)]
                slc_v[pl.ds(b * _L, _L)] = acc
            pltpu.sync_copy(slc_v, shared_glob.at[pl.ds(sid * _SLC, _SLC)])
        plsc.subcore_barrier()
        pltpu.sync_copy(shared_glob, cnt_v)

        def kloop(b, acc):
            cnt = cnt_v[pl.ds(b * _L, _L)]
            binid = lax.iota(jnp.int32, _L) + b * _L
            live = (cnt > 0.0) & (binid < _NCLU)
            return acc + jnp.where(live, 1.0, 0.0)
        kacc = lax.fori_loop(0, _BINS // _L, kloop,
                             jnp.zeros((_L,), jnp.float32))
        ktot = jnp.full((_L,), jnp.sum(kacc), dtype=jnp.float32)
        kinv = jnp.where(ktot > 0.0, 1.0 / ktot, 0.0)

        def wloop(j, _):
            yv = y_v[pl.ds(j * _L, _L)]
            msk = yv >= 0
            safe = jnp.where(msk, yv, 0)
            cnt = plsc.load_gather(cnt_v, [safe])
            wv = jnp.where(msk, kinv / cnt, 0.0)
            winv_v[pl.ds(j * _L, _L)] = wv
            return _
        lax.fori_loop(0, per // _L, wloop, 0)
        pltpu.sync_copy(winv_v, winv_hbm.at[pl.ds(base, per)])


def _sc_weights(pseudo_label):
    n = pseudo_label.shape[0]
    mesh = plsc.VectorSubcoreMesh(core_axis_name="c", subcore_axis_name="s")
    fn = functools.partial(
        pl.kernel,
        mesh=mesh,
        compiler_params=pltpu.CompilerParams(needs_layout_passes=False),
        out_type=jax.ShapeDtypeStruct((n,), jnp.float32),
        scratch_types=[
            pltpu.VMEM((n // _TILES,), jnp.int32),
            pltpu.VMEM((_BINS,), jnp.float32),
            pltpu.VMEM((n // _TILES,), jnp.float32),
            pltpu.VMEM((_TILES, _SLC), jnp.float32),
            pltpu.VMEM((_SLC,), jnp.float32),
            pltpu.VMEM_SHARED((_TILES, _BINS), jnp.float32),
            pltpu.VMEM_SHARED((_BINS,), jnp.float32),
        ],
    )(_sc_body)
    return fn(pseudo_label)


def _tc_body(c_ref, y_ref, w_ref, out_ref):
    i = pl.program_id(0)
    x = c_ref[...]
    r, cnum = x.shape
    m = jnp.max(x, axis=1, keepdims=True)
    se = jnp.sum(jnp.exp(x - m), axis=1, keepdims=True)
    lse = jnp.log(se) + m
    y = y_ref[...]
    col = lax.broadcasted_iota(jnp.int32, (r, cnum), 1)
    oh = col == y
    picked = jnp.sum(jnp.where(oh, x, 0.0), axis=1, keepdims=True)
    nll = lse - picked
    part = jnp.sum(w_ref[...] * nll)

    @pl.when(i == 0)
    def _():
        out_ref[...] = jnp.zeros_like(out_ref)

    out_ref[...] += jnp.full((1, 1), part, dtype=jnp.float32)


def kernel(c, pseudo_label):
    n, cnum = c.shape
    y2d = pseudo_label.reshape(n, 1).astype(jnp.int32)
    winv = _sc_weights(pseudo_label.astype(jnp.int32))
    winv2d = winv.reshape(n, 1)
    out = pl.pallas_call(
        _tc_body,
        grid=(n // _R,),
        compiler_params=pltpu.CompilerParams(
            allow_input_fusion=[True, False, False]),
        in_specs=[
            pl.BlockSpec((_R, cnum), lambda i: (i, 0)),
            pl.BlockSpec((_R, 1), lambda i: (i, 0)),
            pl.BlockSpec((_R, 1), lambda i: (i, 0)),
        ],
        out_specs=pl.BlockSpec((1, 1), lambda i: (0, 0)),
        out_shape=jax.ShapeDtypeStruct((1, 1), jnp.float32),
    )(c, y2d, winv2d)
    return out[0, 0]
